# baseline (speedup 1.0000x reference)
import jax
import jax.numpy as jnp
from jax import lax
from jax.experimental import pallas as pl
from jax.experimental.pallas import tpu as pltpu
from jax.experimental.pallas import tpu_sc as plsc

_B = 65536
_L = 16
_NC = 1
_NS = 16
_NW = _NC * _NS
_CHUNK = _B // _NW
_H = _CHUNK // 2


def _compute_half(op_v, a_v, b_v, c_v, o_v, lo):
    @plsc.parallel_loop(lo, lo + _H, _L, unroll=2)
    def step(j):
        off = pl.multiple_of(j, _L)
        op = op_v[pl.ds(off, _L)]
        a = a_v[pl.ds(off, _L)]
        b = b_v[pl.ds(off, _L)]
        c = c_v[pl.ds(off, _L)]
        t = jnp.where(op == 0, b + c,
                      jnp.where(op == 4, a,
                                jnp.where(op == 6, 1, -1)))
        r_arith = (a + t) & 255
        r_bit = jnp.where(op == 1, a & b,
                          jnp.where(op == 2, a | b, a ^ b))
        is_bit = (op >= 1) & (op <= 3)
        res = jnp.where(is_bit, r_bit,
                        jnp.where(op == 5, a >> 1, r_arith))
        o_v[pl.ds(off, _L)] = res


def _body(op_hbm, a_hbm, b_hbm, c_hbm, out_hbm,
          op_v, a_v, b_v, c_v, o_v, sem0, sem1, sem_out):
    wid = lax.axis_index("s") * _NC + lax.axis_index("c")
    base = wid * _CHUNK
    sems = (sem0, sem1)
    in_cps = []
    for h in range(2):
        hbm_sl = pl.ds(base + h * _H, _H)
        vm_sl = pl.ds(h * _H, _H)
        in_cps.append([
            pltpu.async_copy(src.at[hbm_sl], dst.at[vm_sl], sems[h])
            for src, dst in ((op_hbm, op_v), (a_hbm, a_v),
                             (b_hbm, b_v), (c_hbm, c_v))])
    out_cps = []
    for h in range(2):
        for cp in in_cps[h]:
            cp.wait()
        _compute_half(op_v, a_v, b_v, c_v, o_v, h * _H)
        out_cps.append(pltpu.async_copy(
            o_v.at[pl.ds(h * _H, _H)],
            out_hbm.at[pl.ds(base + h * _H, _H)], sem_out))
    for cp in out_cps:
        cp.wait()


def kernel(opcode, a, b, carry, emb_table, signatures, atom_positions,
           composition_table):
    del emb_table, signatures, atom_positions, composition_table
    mesh = plsc.VectorSubcoreMesh(core_axis_name="c", subcore_axis_name="s",
                                  num_cores=1)
    f = pl.kernel(
        _body,
        mesh=mesh,
        out_type=jax.ShapeDtypeStruct((_B,), jnp.int32),
        scratch_types=[pltpu.VMEM((_CHUNK,), jnp.int32) for _ in range(5)]
        + [pltpu.SemaphoreType.DMA for _ in range(3)],
    )
    return f(opcode, a, b, carry)


# speedup vs baseline: 1.0074x; 1.0074x over previous
import jax
import jax.numpy as jnp
from jax import lax
from jax.experimental import pallas as pl
from jax.experimental.pallas import tpu as pltpu
from jax.experimental.pallas import tpu_sc as plsc

_B = 65536
_L = 16
_NC = 1
_NS = 16
_NW = _NC * _NS
_CHUNK = _B // _NW
_H = _CHUNK // 2


def _compute_half(op_v, a_v, b_v, c_v, o_v, lo):
    @plsc.parallel_loop(lo, lo + _H, _L, unroll=1)
    def step(j):
        off = pl.multiple_of(j, _L)
        op = op_v[pl.ds(off, _L)]
        a = a_v[pl.ds(off, _L)]
        b = b_v[pl.ds(off, _L)]
        c = c_v[pl.ds(off, _L)]
        t = jnp.where(op == 0, b + c,
                      jnp.where(op == 4, a,
                                jnp.where(op == 6, 1, -1)))
        r_arith = (a + t) & 255
        r_bit = jnp.where(op == 1, a & b,
                          jnp.where(op == 2, a | b, a ^ b))
        is_bit = (op >= 1) & (op <= 3)
        res = jnp.where(is_bit, r_bit,
                        jnp.where(op == 5, a >> 1, r_arith))
        o_v[pl.ds(off, _L)] = res


def _body(op_hbm, a_hbm, b_hbm, c_hbm, out_hbm,
          op_v, a_v, b_v, c_v, o_v, sem0, sem1, sem_out):
    wid = lax.axis_index("s") * _NC + lax.axis_index("c")
    base = wid * _CHUNK
    sems = (sem0, sem1)
    in_cps = []
    for h in range(2):
        hbm_sl = pl.ds(base + h * _H, _H)
        vm_sl = pl.ds(h * _H, _H)
        in_cps.append([
            pltpu.async_copy(src.at[hbm_sl], dst.at[vm_sl], sems[h])
            for src, dst in ((op_hbm, op_v), (a_hbm, a_v),
                             (b_hbm, b_v), (c_hbm, c_v))])
    out_cps = []
    for h in range(2):
        for cp in in_cps[h]:
            cp.wait()
        _compute_half(op_v, a_v, b_v, c_v, o_v, h * _H)
        out_cps.append(pltpu.async_copy(
            o_v.at[pl.ds(h * _H, _H)],
            out_hbm.at[pl.ds(base + h * _H, _H)], sem_out))
    for cp in out_cps:
        cp.wait()


def kernel(opcode, a, b, carry, emb_table, signatures, atom_positions,
           composition_table):
    del emb_table, signatures, atom_positions, composition_table
    mesh = plsc.VectorSubcoreMesh(core_axis_name="c", subcore_axis_name="s",
                                  num_cores=1)
    f = pl.kernel(
        _body,
        mesh=mesh,
        out_type=jax.ShapeDtypeStruct((_B,), jnp.int32),
        scratch_types=[pltpu.VMEM((_CHUNK,), jnp.int32) for _ in range(5)]
        + [pltpu.SemaphoreType.DMA for _ in range(3)],
    )
    return f(opcode, a, b, carry)
